# baseline (speedup 1.0000x reference)
import functools

import jax
import jax.numpy as jnp
from jax import lax
from jax.experimental import pallas as pl
from jax.experimental.pallas import tpu as pltpu
from jax.experimental.pallas import tpu_sc as plsc

_CH = 128


def _sc_gather(table, idx, B, D, nw):
    bpw = B // nw
    nch = bpw // _CH

    @functools.partial(
        pl.kernel,
        mesh=plsc.VectorSubcoreMesh(core_axis_name="c", subcore_axis_name="s"),
        out_type=jax.ShapeDtypeStruct((B, D), jnp.float32),
        scratch_types=[
            pltpu.VMEM((nch, _CH), jnp.int32),
            pltpu.VMEM((bpw, D), jnp.float32),
            pltpu.SemaphoreType.DMA,
        ],
        compiler_params=pltpu.CompilerParams(use_tc_tiling_on_sc=False),
    )
    def gather_k(table_hbm, idx_hbm, out_hbm, idx_v, rows_v, sem):
        nc = 2
        wid = lax.axis_index("s") * nc + lax.axis_index("c")
        base = pl.multiple_of(wid * bpw, bpw)
        for j in range(nch):
            pltpu.sync_copy(idx_hbm.at[pl.ds(base + j * _CH, _CH)],
                            idx_v.at[j])
        copies = [
            pltpu.async_copy(
                table_hbm.at[idx_v.at[j]],
                rows_v.at[pl.ds(j * _CH, _CH)],
                sem,
            )
            for j in range(nch)
        ]
        for cp in copies:
            cp.wait()
        pltpu.sync_copy(rows_v, out_hbm.at[pl.ds(base, bpw)])

    return gather_k(table, idx)


def _head_body(x_ref, w0x_ref, b0_ref, a_ref):
    a_ref[...] = jnp.dot(x_ref[...], w0x_ref[...],
                         preferred_element_type=jnp.float32,
                         precision=lax.Precision.HIGHEST) + b0_ref[...]


def _tail_body(a_ref, e_ref, w0e_ref, w1_ref, b1_ref,
               w2_ref, b2_ref, wt_ref, bt_ref, o_ref):
    f32 = jnp.float32
    hi = lax.Precision.HIGHEST
    h = a_ref[...] + jnp.dot(e_ref[...], w0e_ref[...],
                             preferred_element_type=f32, precision=hi)
    h = jnp.maximum(h, 0.0)
    h = h + jnp.maximum(
        jnp.dot(h, w1_ref[...], preferred_element_type=f32,
                precision=hi) + b1_ref[...], 0.0)
    h = h + jnp.maximum(
        jnp.dot(h, w2_ref[...], preferred_element_type=f32,
                precision=hi) + b2_ref[...], 0.0)
    o_ref[...] = jnp.dot(h, wt_ref[...], preferred_element_type=f32,
                         precision=hi) + bt_ref[...]


def kernel(x, table, W0, b0, W1, b1, W2, b2, Wt, bt):
    B, C = x.shape
    D = table.shape[1]
    H = W0.shape[1]
    HP = 128

    info = plsc.get_sparse_core_info()
    nw = info.num_cores * info.num_subcores

    idx = x[:, 0].astype(jnp.int32)
    emb = _sc_gather(table, idx, B, D, nw)

    ph = HP - H
    w0x = jnp.pad(W0[D:], ((1, 0), (0, ph)))
    w0e = jnp.pad(W0[:D], ((0, 0), (0, ph)))
    w1 = jnp.pad(W1, ((0, ph), (0, ph)))
    w2 = jnp.pad(W2, ((0, ph), (0, ph)))
    wt = jnp.pad(Wt, ((0, ph), (0, 0)))
    b0p = jnp.pad(b0, (0, ph)).reshape(1, HP)
    b1p = jnp.pad(b1, (0, ph)).reshape(1, HP)
    b2p = jnp.pad(b2, (0, ph)).reshape(1, HP)
    btp = bt.reshape(1, 1)

    BB = 2048
    nb = B // BB
    rep = lambda i: (0, 0)

    a0 = pl.pallas_call(
        _head_body,
        grid=(nb,),
        in_specs=[
            pl.BlockSpec((BB, C), lambda i: (i, 0)),
            pl.BlockSpec((C, HP), rep),
            pl.BlockSpec((1, HP), rep),
        ],
        out_specs=pl.BlockSpec((BB, HP), lambda i: (i, 0)),
        out_shape=jax.ShapeDtypeStruct((B, HP), jnp.float32),
    )(x, w0x, b0p)

    out = pl.pallas_call(
        _tail_body,
        grid=(nb,),
        in_specs=[
            pl.BlockSpec((BB, HP), lambda i: (i, 0)),
            pl.BlockSpec((BB, D), lambda i: (i, 0)),
            pl.BlockSpec((D, HP), rep),
            pl.BlockSpec((HP, HP), rep),
            pl.BlockSpec((1, HP), rep),
            pl.BlockSpec((HP, HP), rep),
            pl.BlockSpec((1, HP), rep),
            pl.BlockSpec((HP, 1), rep),
            pl.BlockSpec((1, 1), rep),
        ],
        out_specs=pl.BlockSpec((BB, 1), lambda i: (i, 0)),
        out_shape=jax.ShapeDtypeStruct((B, 1), jnp.float32),
    )(a0, emb, w0e, w1, b1p, w2, b2p, wt, btp)
    return out


# speedup vs baseline: 2.1931x; 2.1931x over previous
import functools

import jax
import jax.numpy as jnp
from jax import lax
from jax.experimental import pallas as pl
from jax.experimental.pallas import tpu as pltpu
from jax.experimental.pallas import tpu_sc as plsc

_NSLOT = 4


def _sc_gather(table_t, idx, B, D, nw):
    bpw = B // nw
    ngrp = bpw // 16

    @functools.partial(
        pl.kernel,
        mesh=plsc.VectorSubcoreMesh(core_axis_name="c", subcore_axis_name="s"),
        out_type=jax.ShapeDtypeStruct((D, B), jnp.float32),
        scratch_types=[
            pltpu.VMEM((bpw,), jnp.int32),
            pltpu.VMEM((_NSLOT, D, 128), jnp.float32),
            pltpu.VMEM((D, bpw), jnp.float32),
            [pltpu.SemaphoreType.DMA] * _NSLOT,
        ],
        compiler_params=pltpu.CompilerParams(
            disable_bounds_checks=True, needs_layout_passes=False),
    )
    def gather_k(table_hbm, idx_hbm, out_hbm, idx_v, slabs_v, embt_v, sems):
        nc = 2
        wid = lax.axis_index("s") * nc + lax.axis_index("c")
        base = pl.multiple_of(wid * bpw, bpw)
        pltpu.sync_copy(idx_hbm.at[pl.ds(base, bpw)], idx_v)

        lanes = lax.iota(jnp.int32, 16)

        def slab_copy(i, slot):
            off = pl.multiple_of((i >> 7) * 128, 128)
            pltpu.async_copy(
                table_hbm.at[:, pl.ds(off, 128)], slabs_v.at[slot],
                sems[slot])

        def slab_wait(slot):
            pltpu.make_async_copy(
                table_hbm.at[:, pl.ds(0, 128)], slabs_v.at[slot],
                sems[slot]).wait()

        def group(g, _):
            v = idx_v[pl.ds(g * 16, 16)]
            scalars = [v[l] for l in range(16)]
            for l in range(_NSLOT):
                slab_copy(scalars[l], l)
            for l in range(16):
                slot = l % _NSLOT
                slab_wait(slot)
                r = scalars[l] & 127
                rvec = jnp.full((16,), r, jnp.int32)
                col = g * 16 + l
                cvec = jnp.full((16,), col, jnp.int32)
                for t in range(D // 16):
                    c0 = lanes + (16 * t)
                    vals = plsc.load_gather(slabs_v.at[slot], [c0, rvec])
                    plsc.store_scatter(embt_v, [c0, cvec], vals)
                if l + _NSLOT < 16:
                    slab_copy(scalars[l + _NSLOT], slot)
            return 0

        lax.fori_loop(0, ngrp, group, 0)
        pltpu.sync_copy(embt_v, out_hbm.at[:, pl.ds(base, bpw)])

    return gather_k(table_t, idx)


def _head_body(x_ref, w0x_ref, b0_ref, a_ref):
    a_ref[...] = jnp.dot(x_ref[...], w0x_ref[...],
                         preferred_element_type=jnp.float32,
                         precision=lax.Precision.HIGHEST) + b0_ref[...]


def _tail_body(a_ref, et_ref, w0e_ref, w1_ref, b1_ref,
               w2_ref, b2_ref, wt_ref, bt_ref, o_ref):
    f32 = jnp.float32
    hi = lax.Precision.HIGHEST
    h = a_ref[...] + lax.dot_general(et_ref[...], w0e_ref[...],
                                     (((0,), (0,)), ((), ())),
                                     preferred_element_type=f32, precision=hi)
    h = jnp.maximum(h, 0.0)
    h = h + jnp.maximum(
        jnp.dot(h, w1_ref[...], preferred_element_type=f32,
                precision=hi) + b1_ref[...], 0.0)
    h = h + jnp.maximum(
        jnp.dot(h, w2_ref[...], preferred_element_type=f32,
                precision=hi) + b2_ref[...], 0.0)
    o_ref[...] = jnp.dot(h, wt_ref[...], preferred_element_type=f32,
                         precision=hi) + bt_ref[...]


def kernel(x, table, W0, b0, W1, b1, W2, b2, Wt, bt):
    B, C = x.shape
    D = table.shape[1]
    H = W0.shape[1]
    HP = 128

    info = plsc.get_sparse_core_info()
    nw = info.num_cores * info.num_subcores

    idx = x[:, 0].astype(jnp.int32)
    embt = _sc_gather(table.T, idx, B, D, nw)

    ph = HP - H
    w0x = jnp.pad(W0[D:], ((1, 0), (0, ph)))
    w0e = jnp.pad(W0[:D], ((0, 0), (0, ph)))
    w1 = jnp.pad(W1, ((0, ph), (0, ph)))
    w2 = jnp.pad(W2, ((0, ph), (0, ph)))
    wt = jnp.pad(Wt, ((0, ph), (0, 0)))
    b0p = jnp.pad(b0, (0, ph)).reshape(1, HP)
    b1p = jnp.pad(b1, (0, ph)).reshape(1, HP)
    b2p = jnp.pad(b2, (0, ph)).reshape(1, HP)
    btp = bt.reshape(1, 1)

    BB = 2048
    nb = B // BB
    rep = lambda i: (0, 0)

    a0 = pl.pallas_call(
        _head_body,
        grid=(nb,),
        in_specs=[
            pl.BlockSpec((BB, C), lambda i: (i, 0)),
            pl.BlockSpec((C, HP), rep),
            pl.BlockSpec((1, HP), rep),
        ],
        out_specs=pl.BlockSpec((BB, HP), lambda i: (i, 0)),
        out_shape=jax.ShapeDtypeStruct((B, HP), jnp.float32),
    )(x, w0x, b0p)

    out = pl.pallas_call(
        _tail_body,
        grid=(nb,),
        in_specs=[
            pl.BlockSpec((BB, HP), lambda i: (i, 0)),
            pl.BlockSpec((D, BB), lambda i: (0, i)),
            pl.BlockSpec((D, HP), rep),
            pl.BlockSpec((HP, HP), rep),
            pl.BlockSpec((1, HP), rep),
            pl.BlockSpec((HP, HP), rep),
            pl.BlockSpec((1, HP), rep),
            pl.BlockSpec((HP, 1), rep),
            pl.BlockSpec((1, 1), rep),
        ],
        out_specs=pl.BlockSpec((BB, 1), lambda i: (i, 0)),
        out_shape=jax.ShapeDtypeStruct((B, 1), jnp.float32),
    )(a0, embt, w0e, w1, b1p, w2, b2p, wt, btp)
    return out


# speedup vs baseline: 2.9371x; 1.3393x over previous
import functools

import jax
import jax.numpy as jnp
from jax import lax
from jax.experimental import pallas as pl
from jax.experimental.pallas import tpu as pltpu
from jax.experimental.pallas import tpu_sc as plsc

_NSLOT = 8


def _sc_gather(table_t, idx, B, D, nw):
    bpw = B // nw
    ngrp = bpw // 16

    @functools.partial(
        pl.kernel,
        mesh=plsc.VectorSubcoreMesh(core_axis_name="c", subcore_axis_name="s"),
        out_type=jax.ShapeDtypeStruct((D, B), jnp.float32),
        scratch_types=[
            pltpu.VMEM((bpw,), jnp.int32),
            pltpu.VMEM((_NSLOT, D, 128), jnp.float32),
            pltpu.VMEM((D, bpw), jnp.float32),
            [pltpu.SemaphoreType.DMA] * _NSLOT,
        ],
        compiler_params=pltpu.CompilerParams(
            disable_bounds_checks=True, needs_layout_passes=False),
    )
    def gather_k(table_hbm, idx_hbm, out_hbm, idx_v, slabs_v, embt_v, sems):
        nc = 2
        wid = lax.axis_index("s") * nc + lax.axis_index("c")
        base = pl.multiple_of(wid * bpw, bpw)
        pltpu.sync_copy(idx_hbm.at[pl.ds(base, bpw)], idx_v)

        lanes = lax.iota(jnp.int32, 16)

        def slab_copy(i, slot):
            off = pl.multiple_of((i >> 7) * 128, 128)
            pltpu.async_copy(
                table_hbm.at[:, pl.ds(off, 128)], slabs_v.at[slot],
                sems[slot])

        def slab_wait(slot):
            pltpu.make_async_copy(
                table_hbm.at[:, pl.ds(0, 128)], slabs_v.at[slot],
                sems[slot]).wait()

        v0 = idx_v[pl.ds(0, 16)]
        for l in range(_NSLOT):
            slab_copy(v0[l], l)

        def group(g, _):
            v = idx_v[pl.ds(g * 16, 16)]
            g1 = jnp.minimum(g + 1, ngrp - 1)
            vn = idx_v[pl.ds(g1 * 16, 16)]
            scalars = [v[l] for l in range(16)]
            nscalars = [vn[l] for l in range(16)]
            ahead = scalars[_NSLOT:] + nscalars[:_NSLOT]
            for l in range(16):
                slot = l % _NSLOT
                slab_wait(slot)
                r = scalars[l] & 127
                rvec = jnp.full((16,), r, jnp.int32)
                col = g * 16 + l
                cvec = jnp.full((16,), col, jnp.int32)
                for t in range(D // 16):
                    c0 = lanes + (16 * t)
                    vals = plsc.load_gather(slabs_v.at[slot], [c0, rvec])
                    plsc.store_scatter(embt_v, [c0, cvec], vals)
                slab_copy(ahead[l], slot)
            return 0

        lax.fori_loop(0, ngrp, group, 0)
        for s in range(_NSLOT):
            slab_wait(s)
        pltpu.sync_copy(embt_v, out_hbm.at[:, pl.ds(base, bpw)])

    return gather_k(table_t, idx)


def _head_body(x_ref, w0x_ref, b0_ref, a_ref):
    a_ref[...] = jnp.dot(x_ref[...], w0x_ref[...],
                         preferred_element_type=jnp.float32,
                         precision=lax.Precision.DEFAULT) + b0_ref[...]


def _tail_body(a_ref, et_ref, w0e_ref, w1_ref, b1_ref,
               w2_ref, b2_ref, wt_ref, bt_ref, o_ref):
    f32 = jnp.float32
    hi = lax.Precision.DEFAULT
    h = a_ref[...] + lax.dot_general(et_ref[...], w0e_ref[...],
                                     (((0,), (0,)), ((), ())),
                                     preferred_element_type=f32, precision=hi)
    h = jnp.maximum(h, 0.0)
    h = h + jnp.maximum(
        jnp.dot(h, w1_ref[...], preferred_element_type=f32,
                precision=hi) + b1_ref[...], 0.0)
    h = h + jnp.maximum(
        jnp.dot(h, w2_ref[...], preferred_element_type=f32,
                precision=hi) + b2_ref[...], 0.0)
    o_ref[...] = jnp.dot(h, wt_ref[...], preferred_element_type=f32,
                         precision=hi) + bt_ref[...]


def kernel(x, table, W0, b0, W1, b1, W2, b2, Wt, bt):
    B, C = x.shape
    D = table.shape[1]
    H = W0.shape[1]
    HP = 128

    info = plsc.get_sparse_core_info()
    nw = info.num_cores * info.num_subcores

    idx = x[:, 0].astype(jnp.int32)
    embt = _sc_gather(table.T, idx, B, D, nw)

    ph = HP - H
    w0x = jnp.pad(W0[D:], ((1, 0), (0, ph)))
    w0e = jnp.pad(W0[:D], ((0, 0), (0, ph)))
    w1 = jnp.pad(W1, ((0, ph), (0, ph)))
    w2 = jnp.pad(W2, ((0, ph), (0, ph)))
    wt = jnp.pad(Wt, ((0, ph), (0, 0)))
    b0p = jnp.pad(b0, (0, ph)).reshape(1, HP)
    b1p = jnp.pad(b1, (0, ph)).reshape(1, HP)
    b2p = jnp.pad(b2, (0, ph)).reshape(1, HP)
    btp = bt.reshape(1, 1)

    BB = 2048
    nb = B // BB
    rep = lambda i: (0, 0)

    a0 = pl.pallas_call(
        _head_body,
        grid=(nb,),
        in_specs=[
            pl.BlockSpec((BB, C), lambda i: (i, 0)),
            pl.BlockSpec((C, HP), rep),
            pl.BlockSpec((1, HP), rep),
        ],
        out_specs=pl.BlockSpec((BB, HP), lambda i: (i, 0)),
        out_shape=jax.ShapeDtypeStruct((B, HP), jnp.float32),
    )(x, w0x, b0p)

    out = pl.pallas_call(
        _tail_body,
        grid=(nb,),
        in_specs=[
            pl.BlockSpec((BB, HP), lambda i: (i, 0)),
            pl.BlockSpec((D, BB), lambda i: (0, i)),
            pl.BlockSpec((D, HP), rep),
            pl.BlockSpec((HP, HP), rep),
            pl.BlockSpec((1, HP), rep),
            pl.BlockSpec((HP, HP), rep),
            pl.BlockSpec((1, HP), rep),
            pl.BlockSpec((HP, 1), rep),
            pl.BlockSpec((1, 1), rep),
        ],
        out_specs=pl.BlockSpec((BB, 1), lambda i: (i, 0)),
        out_shape=jax.ShapeDtypeStruct((B, 1), jnp.float32),
    )(a0, embt, w0e, w1, b1p, w2, b2p, wt, btp)
    return out
